# Initial kernel scaffold; baseline (speedup 1.0000x reference)
import functools

import jax
import jax.numpy as jnp
from jax import lax
from jax.experimental import pallas as pl
from jax.experimental.pallas import tpu as pltpu
from jax.experimental.pallas import tpu_sc as plsc

EMB = 64
NC = 2
NS = 16
NW = NC * NS
CH = 128
K = 4
SUP = CH * K
NBUF = 2


@functools.lru_cache(maxsize=None)
def _build(n_total, vocab):
    per_w = n_total // NW
    nch = per_w // CH
    nsup = nch // K
    ngrp = nsup // NBUF

    mesh = plsc.VectorSubcoreMesh(core_axis_name="c", subcore_axis_name="s")

    @functools.partial(
        pl.kernel,
        mesh=mesh,
        out_type=jax.ShapeDtypeStruct((n_total, EMB), jnp.float32),
        scratch_types=[
            pltpu.VMEM((nch, CH), jnp.int32),
            pltpu.VMEM((NBUF, SUP, EMB), jnp.float32),
            pltpu.SemaphoreType.DMA((NBUF,)),
            pltpu.SemaphoreType.DMA((NBUF,)),
        ],
    )
    def k(idx_hbm, table_hbm, out_hbm, idx_v, rows_v, gsem, ssem):
        wid = lax.axis_index("s") * NC + lax.axis_index("c")
        base = wid * per_w

        pltpu.sync_copy(idx_hbm.at[wid], idx_v)

        def start_gathers(t, b):
            for u in range(K):
                j = t * K + u
                pltpu.async_copy(
                    table_hbm.at[idx_v.at[j]],
                    rows_v.at[b, pl.ds(u * CH, CH)],
                    gsem.at[b],
                )

        def wait_gathers(b):
            pltpu.make_async_copy(
                table_hbm.at[pl.ds(0, SUP)], rows_v.at[b], gsem.at[b]
            ).wait()

        def start_store(t, b):
            pltpu.async_copy(
                rows_v.at[b], out_hbm.at[pl.ds(base + t * SUP, SUP)], ssem.at[b]
            )

        def wait_store(b):
            pltpu.make_async_copy(
                rows_v.at[b], out_hbm.at[pl.ds(base, SUP)], ssem.at[b]
            ).wait()

        for b in range(NBUF):
            start_gathers(b, b)

        def body(g, carry):
            for b in range(NBUF):
                t = g * NBUF + b
                wait_gathers(b)
                start_store(t, b)
            for b in range(NBUF):
                t2 = g * NBUF + b + NBUF

                @pl.when(t2 < nsup)
                def _():
                    wait_store(b)
                    start_gathers(t2, b)

            return carry

        lax.fori_loop(0, ngrp, body, 0)

        for b in range(NBUF):
            wait_store(b)

    return k


def kernel(indices, table):
    bsz, seq = indices.shape
    n_total = bsz * seq
    vocab, emb = table.shape
    idx = indices.reshape(NW, n_total // (NW * CH), CH).astype(jnp.int32)
    out = _build(n_total, vocab)(idx, table)
    return out.reshape(bsz, seq, emb)


# speedup vs baseline: 4.2122x; 4.2122x over previous
import functools

import jax
import jax.numpy as jnp
from jax import lax
from jax.experimental import pallas as pl
from jax.experimental.pallas import tpu as pltpu
from jax.experimental.pallas import tpu_sc as plsc

EMB = 64
NC = 2
NS = 16
NW = NC * NS
CH = 128
K = 4
SUP = CH * K
NBUF = 2


@functools.lru_cache(maxsize=None)
def _build(n_total, vocab):
    per_w = n_total // NW
    nch = per_w // CH
    nsup = nch // K
    ngrp = nsup // NBUF

    mesh = plsc.VectorSubcoreMesh(core_axis_name="c", subcore_axis_name="s")

    @functools.partial(
        pl.kernel,
        mesh=mesh,
        out_type=jax.ShapeDtypeStruct((n_total, EMB), jnp.float32),
        scratch_types=[
            pltpu.VMEM((nch, CH), jnp.int32),
            pltpu.VMEM((NBUF, SUP, EMB), jnp.float32),
            pltpu.SemaphoreType.DMA((NBUF,)),
            pltpu.SemaphoreType.DMA((NBUF,)),
        ],
        compiler_params=pltpu.CompilerParams(use_tc_tiling_on_sc=False),
    )
    def k(idx_hbm, table_hbm, out_hbm, idx_v, rows_v, gsem, ssem):
        wid = lax.axis_index("s") * NC + lax.axis_index("c")
        base = wid * per_w

        pltpu.sync_copy(idx_hbm.at[wid], idx_v)

        def start_gathers(t, b):
            for u in range(K):
                j = t * K + u
                pltpu.async_copy(
                    table_hbm.at[idx_v.at[j]],
                    rows_v.at[b, pl.ds(u * CH, CH)],
                    gsem.at[b],
                )

        def wait_gathers(b):
            pltpu.make_async_copy(
                table_hbm.at[pl.ds(0, SUP)], rows_v.at[b], gsem.at[b]
            ).wait()

        def start_store(t, b):
            pltpu.async_copy(
                rows_v.at[b], out_hbm.at[pl.ds(base + t * SUP, SUP)], ssem.at[b]
            )

        def wait_store(b):
            pltpu.make_async_copy(
                rows_v.at[b], out_hbm.at[pl.ds(base, SUP)], ssem.at[b]
            ).wait()

        for b in range(NBUF):
            start_gathers(b, b)

        def body(g, carry):
            for b in range(NBUF):
                t = g * NBUF + b
                wait_gathers(b)
                start_store(t, b)
            for b in range(NBUF):
                t2 = g * NBUF + b + NBUF

                @pl.when(t2 < nsup)
                def _():
                    wait_store(b)
                    start_gathers(t2, b)

            return carry

        lax.fori_loop(0, ngrp, body, 0)

        for b in range(NBUF):
            wait_store(b)

    return k


def kernel(indices, table):
    bsz, seq = indices.shape
    n_total = bsz * seq
    vocab, emb = table.shape
    idx = indices.reshape(NW, n_total // (NW * CH), CH).astype(jnp.int32)
    out = _build(n_total, vocab)(idx, table)
    return out.reshape(bsz, seq, emb)


# speedup vs baseline: 4.2271x; 1.0035x over previous
import functools

import jax
import jax.numpy as jnp
from jax import lax
from jax.experimental import pallas as pl
from jax.experimental.pallas import tpu as pltpu
from jax.experimental.pallas import tpu_sc as plsc

EMB = 64
NC = 2
NS = 16
NW = NC * NS
CH = 128
K = 4
SUP = CH * K
NBUF = 2


@functools.lru_cache(maxsize=None)
def _build(bsz, seq, vocab):
    n_total = bsz * seq
    per_w = n_total // NW
    nch = per_w // CH
    nsup = nch // K
    ngrp = nsup // NBUF
    sup_per_seq = seq // SUP

    mesh = plsc.VectorSubcoreMesh(core_axis_name="c", subcore_axis_name="s")

    @functools.partial(
        pl.kernel,
        mesh=mesh,
        out_type=jax.ShapeDtypeStruct((bsz, seq, EMB), jnp.float32),
        scratch_types=[
            pltpu.VMEM((nch, CH), jnp.int32),
            pltpu.VMEM((NBUF, SUP, EMB), jnp.float32),
            pltpu.SemaphoreType.DMA((NBUF,)),
            pltpu.SemaphoreType.DMA((NBUF,)),
        ],
        compiler_params=pltpu.CompilerParams(use_tc_tiling_on_sc=False),
    )
    def k(idx_hbm, table_hbm, out_hbm, idx_v, rows_v, gsem, ssem):
        wid = lax.axis_index("s") * NC + lax.axis_index("c")
        base_sup = wid * nsup

        pltpu.sync_copy(idx_hbm.at[wid], idx_v)

        def start_gathers(t, b):
            for u in range(K):
                j = t * K + u
                pltpu.async_copy(
                    table_hbm.at[idx_v.at[j]],
                    rows_v.at[b, pl.ds(u * CH, CH)],
                    gsem.at[b],
                )

        def wait_gathers(b):
            pltpu.make_async_copy(
                table_hbm.at[pl.ds(0, SUP)], rows_v.at[b], gsem.at[b]
            ).wait()

        def start_store(t, b):
            g = base_sup + t
            pltpu.async_copy(
                rows_v.at[b],
                out_hbm.at[g // sup_per_seq, pl.ds((g % sup_per_seq) * SUP, SUP)],
                ssem.at[b],
            )

        def wait_store(b):
            pltpu.make_async_copy(
                rows_v.at[b], out_hbm.at[0, pl.ds(0, SUP)], ssem.at[b]
            ).wait()

        for b in range(NBUF):
            start_gathers(b, b)

        def body(g, carry):
            for b in range(NBUF):
                t = g * NBUF + b
                wait_gathers(b)
                start_store(t, b)
            for b in range(NBUF):
                t2 = g * NBUF + b + NBUF

                @pl.when(t2 < nsup)
                def _():
                    wait_store(b)
                    start_gathers(t2, b)

            return carry

        lax.fori_loop(0, ngrp, body, 0)

        for b in range(NBUF):
            wait_store(b)

    return k


def kernel(indices, table):
    bsz, seq = indices.shape
    n_total = bsz * seq
    vocab, emb = table.shape
    idx = indices.reshape(NW, n_total // (NW * CH), CH).astype(jnp.int32)
    return _build(bsz, seq, vocab)(idx, table)
